# baseline (speedup 1.0000x reference)
import functools

import jax
import jax.numpy as jnp
from jax import lax
from jax.experimental import pallas as pl
from jax.experimental.pallas import tpu as pltpu
from jax.experimental.pallas import tpu_sc as plsc

B, C, H, W = 16, 80, 128, 128
HW = H * W
NR = C * H
TOPK = 100
KPAD = 128
BIG = 2**30


def _dense_kernel(cls_ref, scores_ref, rmax_ref, cmax_ref):
    x = cls_ref[0]
    fmap = 1.0 / (1.0 + jnp.exp(-x))
    lane = jax.lax.broadcasted_iota(jnp.int32, (C, H, W), 2)
    left = jnp.where(lane == W - 1, -jnp.inf, pltpu.roll(fmap, W - 1, 2))
    right = jnp.where(lane == 0, -jnp.inf, pltpu.roll(fmap, 1, 2))
    m1 = jnp.maximum(jnp.maximum(left, right), fmap)
    ninf_h = jnp.full((C, 1, W), -jnp.inf, jnp.float32)
    up = jnp.concatenate([m1[:, 1:, :], ninf_h], axis=1)
    down = jnp.concatenate([ninf_h, m1[:, :-1, :]], axis=1)
    fmax = jnp.maximum(jnp.maximum(up, down), m1)
    scores = jnp.where(fmax == fmap, fmap, 0.0)
    scores_ref[0] = scores.reshape(NR, W)
    rmax = jnp.max(scores, axis=2)
    rmax_ref[0] = rmax
    cmax_ref[0] = jnp.max(rmax, axis=1).reshape(1, C)


def _dense_call(cls):
    return pl.pallas_call(
        _dense_kernel,
        grid=(B,),
        in_specs=[pl.BlockSpec((1, C, H, W), lambda b: (b, 0, 0, 0))],
        out_specs=[
            pl.BlockSpec((1, NR, W), lambda b: (b, 0, 0)),
            pl.BlockSpec((1, C, H), lambda b: (b, 0, 0)),
            pl.BlockSpec((1, 1, C), lambda b: (b, 0, 0)),
        ],
        out_shape=[
            jax.ShapeDtypeStruct((B, NR, W), jnp.float32),
            jax.ShapeDtypeStruct((B, C, H), jnp.float32),
            jax.ShapeDtypeStruct((B, 1, C), jnp.float32),
        ],
        compiler_params=pltpu.CompilerParams(
            dimension_semantics=("arbitrary",),
        ),
    )(cls)


def _iota16():
    return jnp.arange(16, dtype=jnp.int32)


def _sc_extract(scores_hbm, rmax_hbm, cmax_hbm, reg_hbm, wh_hbm,
                x1o, y1o, x2o, y2o, sco, clo,
                r_v, cm_v, slot_v, cache_v, vals_v, lidx_v, idx_v,
                g0_v, g1_v, g2_v, g3_v, out_v, sem):
    cid = lax.axis_index("c")
    sid = lax.axis_index("s")
    wid = sid * 2 + cid

    @pl.when(wid < B)
    def _work():
        b = wid
        pltpu.sync_copy(rmax_hbm.at[b], r_v)
        pltpu.sync_copy(cmax_hbm.at[b], cm_v)
        i16 = _iota16()

        def _init(i, carry):
            slot_v[pl.ds(i * 16, 16)] = jnp.full((16,), -1, jnp.int32)
            return carry
        lax.fori_loop(0, NR // 16, _init, 0)
        for q in range(KPAD // 16):
            vals_v[pl.ds(q * 16, 16)] = jnp.zeros((16,), jnp.float32)
            lidx_v[pl.ds(q * 16, 16)] = jnp.zeros((16,), jnp.int32)

        def _argmin_idx(load, nchunk, base, target):
            idx = jnp.full((16,), BIG, jnp.int32)
            for j in range(nchunk):
                cand = jnp.where(load(j) == target, base + j * 16 + i16,
                                 BIG)
                idx = jnp.minimum(idx, cand)
            return lax.reduce_min(idx, axes=(0,))

        def _argmax_chunks(load, nchunk, base):
            m = jnp.full((16,), -jnp.inf, jnp.float32)
            for j in range(nchunk):
                m = jnp.maximum(m, load(j))
            ms = lax.reduce_max(m, axes=(0,))
            return ms, _argmin_idx(load, nchunk, base, ms)

        def _prefetch(c, carry):
            rb = c * H
            _, rc = _argmax_chunks(
                lambda j: r_v[pl.ds(rb + j * 16, 16)], H // 16, rb)
            pltpu.async_copy(scores_hbm.at[b, rc], cache_v.at[c], sem)
            qp = rc // 16
            slot_v[pl.ds(qp * 16, 16)] = jnp.where(
                qp * 16 + i16 == rc, c, slot_v[pl.ds(qp * 16, 16)])
            _, rc2 = _argmax_chunks(
                lambda j: jnp.where(rb + j * 16 + i16 == rc, -jnp.inf,
                                    r_v[pl.ds(rb + j * 16, 16)]),
                H // 16, rb)
            pltpu.async_copy(scores_hbm.at[b, rc2], cache_v.at[C + c], sem)
            qp2 = rc2 // 16
            slot_v[pl.ds(qp2 * 16, 16)] = jnp.where(
                qp2 * 16 + i16 == rc2, C + c, slot_v[pl.ds(qp2 * 16, 16)])
            return carry
        lax.fori_loop(0, C, _prefetch, 0)
        pltpu.make_async_copy(
            scores_hbm.at[b, pl.ds(0, 2 * C)], cache_v.at[pl.ds(0, 2 * C)],
            sem
        ).wait()

        def _body(k, carry):
            m, cstar = _argmax_chunks(
                lambda j: cm_v[pl.ds(j * 16, 16)], C // 16, 0)
            rbase = cstar * H
            rstar = _argmin_idx(
                lambda j: r_v[pl.ds(rbase + j * 16, 16)], H // 16, rbase, m)
            qr = rstar // 16
            s = lax.reduce_max(
                jnp.where(qr * 16 + i16 == rstar,
                          slot_v[pl.ds(qr * 16, 16)], -BIG), axes=(0,))
            t = jnp.where(s < 0, 2 * C + k, s)

            @pl.when(s < 0)
            def _fetch():
                pltpu.sync_copy(scores_hbm.at[b, rstar], cache_v.at[2 * C + k])

            wstar = _argmin_idx(
                lambda j: cache_v[t, pl.ds(j * 16, 16)], W // 16, 0, m)
            newrow = jnp.full((16,), -jnp.inf, jnp.float32)
            for j in range(W // 16):
                ch = cache_v[t, pl.ds(j * 16, 16)]
                ch = jnp.where(j * 16 + i16 == wstar, -1.0, ch)
                cache_v[t, pl.ds(j * 16, 16)] = ch
                newrow = jnp.maximum(newrow, ch)
            newr = lax.reduce_max(newrow, axes=(0,))
            slot_v[pl.ds(qr * 16, 16)] = jnp.where(
                qr * 16 + i16 == rstar, t, slot_v[pl.ds(qr * 16, 16)])
            r_v[pl.ds(qr * 16, 16)] = jnp.where(
                qr * 16 + i16 == rstar, newr, r_v[pl.ds(qr * 16, 16)])
            newcm = jnp.full((16,), -jnp.inf, jnp.float32)
            for j in range(H // 16):
                newcm = jnp.maximum(newcm, r_v[pl.ds(rbase + j * 16, 16)])
            newc = lax.reduce_max(newcm, axes=(0,))
            qc = cstar // 16
            cm_v[pl.ds(qc * 16, 16)] = jnp.where(
                qc * 16 + i16 == cstar, newc, cm_v[pl.ds(qc * 16, 16)])
            qk = k // 16
            vals_v[pl.ds(qk * 16, 16)] = jnp.where(
                qk * 16 + i16 == k, m, vals_v[pl.ds(qk * 16, 16)])
            lidx_v[pl.ds(qk * 16, 16)] = jnp.where(
                qk * 16 + i16 == k, rstar * W + wstar,
                lidx_v[pl.ds(qk * 16, 16)])
            return carry

        lax.fori_loop(0, TOPK, _body, 0)

        def _gather(tab_hbm, chan, dst):
            for q in range(KPAD // 16):
                lv = lidx_v[pl.ds(q * 16, 16)]
                hw = lv % HW
                idx_v[pl.ds(q * 16, 16)] = b * (2 * HW) + chan * HW + hw
            pltpu.async_copy(tab_hbm.at[idx_v], dst, sem).wait()

        _gather(reg_hbm, 0, g0_v)
        _gather(reg_hbm, 1, g1_v)
        _gather(wh_hbm, 0, g2_v)
        _gather(wh_hbm, 1, g3_v)

        for q in range(KPAD // 16):
            sl = pl.ds(q * 16, 16)
            lv = lidx_v[sl]
            hw = lv % HW
            xs = (hw % W).astype(jnp.float32) + g0_v[sl]
            ys = (hw // W).astype(jnp.float32) + g1_v[sl]
            hw0 = g2_v[sl] / 2
            hw1 = g3_v[sl] / 2
            out_v[0, sl] = xs - hw0
            out_v[1, sl] = ys - hw1
            out_v[2, sl] = xs + hw0
            out_v[3, sl] = ys + hw1
        pltpu.sync_copy(out_v.at[0], x1o.at[b])
        pltpu.sync_copy(out_v.at[1], y1o.at[b])
        pltpu.sync_copy(out_v.at[2], x2o.at[b])
        pltpu.sync_copy(out_v.at[3], y2o.at[b])
        pltpu.sync_copy(vals_v, sco.at[b])
        for q in range(KPAD // 16):
            sl = pl.ds(q * 16, 16)
            lidx_v[sl] = lidx_v[sl] // HW
        pltpu.sync_copy(lidx_v, clo.at[b])


def _sc_call(scores, rmaxf, cmaxf, regf, whf):
    mesh = plsc.VectorSubcoreMesh(core_axis_name="c", subcore_axis_name="s")
    fn = functools.partial(
        pl.kernel, mesh=mesh,
        out_type=[jax.ShapeDtypeStruct((B, KPAD), jnp.float32)] * 5
        + [jax.ShapeDtypeStruct((B, KPAD), jnp.int32)],
        scratch_types=[
            pltpu.VMEM((NR,), jnp.float32),
            pltpu.VMEM((C,), jnp.float32),
            pltpu.VMEM((NR + 16,), jnp.int32),
            pltpu.VMEM((2 * C + TOPK, W), jnp.float32),
            pltpu.VMEM((KPAD,), jnp.float32),
            pltpu.VMEM((KPAD,), jnp.int32),
            pltpu.VMEM((KPAD,), jnp.int32),
            pltpu.VMEM((KPAD,), jnp.float32),
            pltpu.VMEM((KPAD,), jnp.float32),
            pltpu.VMEM((KPAD,), jnp.float32),
            pltpu.VMEM((KPAD,), jnp.float32),
            pltpu.VMEM((4, KPAD), jnp.float32),
            pltpu.SemaphoreType.DMA,
        ],
        compiler_params=pltpu.CompilerParams(needs_layout_passes=False),
    )(_sc_extract)
    return fn(scores, rmaxf, cmaxf, regf, whf)


@jax.jit
def kernel(cls, reg, wh):
    scores, rmax, cmax = _dense_call(cls)
    rmaxf = rmax.reshape(B, NR)
    cmaxf = cmax.reshape(B, C)
    regf = reg.reshape(-1)
    whf = wh.reshape(-1)
    x1, y1, x2, y2, sc, cl = _sc_call(scores, rmaxf, cmaxf, regf, whf)
    bboxes = jnp.stack(
        [x1[:, :TOPK], y1[:, :TOPK], x2[:, :TOPK], y2[:, :TOPK]], axis=-1)
    scores_out = sc[:, :TOPK].reshape(-1)
    clses = cl[:, :TOPK].reshape(-1)
    return bboxes, scores_out, clses


# speedup vs baseline: 1.0021x; 1.0021x over previous
import functools

import jax
import jax.numpy as jnp
from jax import lax
from jax.experimental import pallas as pl
from jax.experimental.pallas import tpu as pltpu
from jax.experimental.pallas import tpu_sc as plsc

B, C, H, W = 16, 80, 128, 128
HW = H * W
NR = C * H
TOPK = 100
KPAD = 128
BIG = 2**30


def _dense_kernel(cls_ref, scores_ref, rmax_ref, cmax_ref):
    x = cls_ref[0]
    fmap = 1.0 / (1.0 + jnp.exp(-x))
    ninf = jnp.full((C, H, 1), -jnp.inf, jnp.float32)
    left = jnp.concatenate([fmap[:, :, 1:], ninf], axis=2)
    right = jnp.concatenate([ninf, fmap[:, :, :-1]], axis=2)
    m1 = jnp.maximum(jnp.maximum(left, right), fmap)
    ninf_h = jnp.full((C, 1, W), -jnp.inf, jnp.float32)
    up = jnp.concatenate([m1[:, 1:, :], ninf_h], axis=1)
    down = jnp.concatenate([ninf_h, m1[:, :-1, :]], axis=1)
    fmax = jnp.maximum(jnp.maximum(up, down), m1)
    scores = jnp.where(fmax == fmap, fmap, 0.0)
    scores_ref[0] = scores.reshape(NR, W)
    rmax = jnp.max(scores, axis=2)
    rmax_ref[0] = rmax
    cmax_ref[0] = jnp.max(rmax, axis=1).reshape(1, C)


def _dense_call(cls):
    return pl.pallas_call(
        _dense_kernel,
        grid=(B,),
        in_specs=[pl.BlockSpec((1, C, H, W), lambda b: (b, 0, 0, 0))],
        out_specs=[
            pl.BlockSpec((1, NR, W), lambda b: (b, 0, 0)),
            pl.BlockSpec((1, C, H), lambda b: (b, 0, 0)),
            pl.BlockSpec((1, 1, C), lambda b: (b, 0, 0)),
        ],
        out_shape=[
            jax.ShapeDtypeStruct((B, NR, W), jnp.float32),
            jax.ShapeDtypeStruct((B, C, H), jnp.float32),
            jax.ShapeDtypeStruct((B, 1, C), jnp.float32),
        ],
        compiler_params=pltpu.CompilerParams(
            dimension_semantics=("arbitrary",),
        ),
    )(cls)


def _iota16():
    return jnp.arange(16, dtype=jnp.int32)


def _sc_extract(scores_hbm, rmax_hbm, cmax_hbm, reg_hbm, wh_hbm,
                x1o, y1o, x2o, y2o, sco, clo,
                r_v, cm_v, slot_v, cache_v, vals_v, lidx_v, idx_v,
                g0_v, g1_v, g2_v, g3_v, out_v, sem):
    cid = lax.axis_index("c")
    sid = lax.axis_index("s")
    wid = sid * 2 + cid

    @pl.when(wid < B)
    def _work():
        b = wid
        pltpu.sync_copy(rmax_hbm.at[b], r_v)
        pltpu.sync_copy(cmax_hbm.at[b], cm_v)
        i16 = _iota16()

        def _init(i, carry):
            slot_v[pl.ds(i * 16, 16)] = jnp.full((16,), -1, jnp.int32)
            return carry
        lax.fori_loop(0, NR // 16, _init, 0)
        for q in range(KPAD // 16):
            vals_v[pl.ds(q * 16, 16)] = jnp.zeros((16,), jnp.float32)
            lidx_v[pl.ds(q * 16, 16)] = jnp.zeros((16,), jnp.int32)

        def _argmin_idx(load, nchunk, base, target):
            idx = jnp.full((16,), BIG, jnp.int32)
            for j in range(nchunk):
                cand = jnp.where(load(j) == target, base + j * 16 + i16,
                                 BIG)
                idx = jnp.minimum(idx, cand)
            return lax.reduce_min(idx, axes=(0,))

        def _argmax_chunks(load, nchunk, base):
            m = jnp.full((16,), -jnp.inf, jnp.float32)
            for j in range(nchunk):
                m = jnp.maximum(m, load(j))
            ms = lax.reduce_max(m, axes=(0,))
            return ms, _argmin_idx(load, nchunk, base, ms)

        def _prefetch(c, carry):
            rb = c * H
            _, rc = _argmax_chunks(
                lambda j: r_v[pl.ds(rb + j * 16, 16)], H // 16, rb)
            pltpu.async_copy(scores_hbm.at[b, rc], cache_v.at[c], sem)
            qp = rc // 16
            slot_v[pl.ds(qp * 16, 16)] = jnp.where(
                qp * 16 + i16 == rc, c, slot_v[pl.ds(qp * 16, 16)])
            _, rc2 = _argmax_chunks(
                lambda j: jnp.where(rb + j * 16 + i16 == rc, -jnp.inf,
                                    r_v[pl.ds(rb + j * 16, 16)]),
                H // 16, rb)
            pltpu.async_copy(scores_hbm.at[b, rc2], cache_v.at[C + c], sem)
            qp2 = rc2 // 16
            slot_v[pl.ds(qp2 * 16, 16)] = jnp.where(
                qp2 * 16 + i16 == rc2, C + c, slot_v[pl.ds(qp2 * 16, 16)])
            return carry
        lax.fori_loop(0, C, _prefetch, 0)
        pltpu.make_async_copy(
            scores_hbm.at[b, pl.ds(0, 2 * C)], cache_v.at[pl.ds(0, 2 * C)],
            sem
        ).wait()

        def _body(k, carry):
            m, cstar = _argmax_chunks(
                lambda j: cm_v[pl.ds(j * 16, 16)], C // 16, 0)
            rbase = cstar * H
            rstar = _argmin_idx(
                lambda j: r_v[pl.ds(rbase + j * 16, 16)], H // 16, rbase, m)
            qr = rstar // 16
            s = lax.reduce_max(
                jnp.where(qr * 16 + i16 == rstar,
                          slot_v[pl.ds(qr * 16, 16)], -BIG), axes=(0,))
            t = jnp.where(s < 0, 2 * C + k, s)

            @pl.when(s < 0)
            def _fetch():
                pltpu.sync_copy(scores_hbm.at[b, rstar], cache_v.at[2 * C + k])

            wstar = _argmin_idx(
                lambda j: cache_v[t, pl.ds(j * 16, 16)], W // 16, 0, m)
            newrow = jnp.full((16,), -jnp.inf, jnp.float32)
            for j in range(W // 16):
                ch = cache_v[t, pl.ds(j * 16, 16)]
                ch = jnp.where(j * 16 + i16 == wstar, -1.0, ch)
                cache_v[t, pl.ds(j * 16, 16)] = ch
                newrow = jnp.maximum(newrow, ch)
            newr = lax.reduce_max(newrow, axes=(0,))
            slot_v[pl.ds(qr * 16, 16)] = jnp.where(
                qr * 16 + i16 == rstar, t, slot_v[pl.ds(qr * 16, 16)])
            r_v[pl.ds(qr * 16, 16)] = jnp.where(
                qr * 16 + i16 == rstar, newr, r_v[pl.ds(qr * 16, 16)])
            newcm = jnp.full((16,), -jnp.inf, jnp.float32)
            for j in range(H // 16):
                newcm = jnp.maximum(newcm, r_v[pl.ds(rbase + j * 16, 16)])
            newc = lax.reduce_max(newcm, axes=(0,))
            qc = cstar // 16
            cm_v[pl.ds(qc * 16, 16)] = jnp.where(
                qc * 16 + i16 == cstar, newc, cm_v[pl.ds(qc * 16, 16)])
            qk = k // 16
            vals_v[pl.ds(qk * 16, 16)] = jnp.where(
                qk * 16 + i16 == k, m, vals_v[pl.ds(qk * 16, 16)])
            lidx_v[pl.ds(qk * 16, 16)] = jnp.where(
                qk * 16 + i16 == k, rstar * W + wstar,
                lidx_v[pl.ds(qk * 16, 16)])
            return carry

        lax.fori_loop(0, TOPK, _body, 0)

        def _gather(tab_hbm, chan, dst):
            for q in range(KPAD // 16):
                lv = lidx_v[pl.ds(q * 16, 16)]
                hw = lv % HW
                idx_v[pl.ds(q * 16, 16)] = b * (2 * HW) + chan * HW + hw
            pltpu.async_copy(tab_hbm.at[idx_v], dst, sem).wait()

        _gather(reg_hbm, 0, g0_v)
        _gather(reg_hbm, 1, g1_v)
        _gather(wh_hbm, 0, g2_v)
        _gather(wh_hbm, 1, g3_v)

        for q in range(KPAD // 16):
            sl = pl.ds(q * 16, 16)
            lv = lidx_v[sl]
            hw = lv % HW
            xs = (hw % W).astype(jnp.float32) + g0_v[sl]
            ys = (hw // W).astype(jnp.float32) + g1_v[sl]
            hw0 = g2_v[sl] / 2
            hw1 = g3_v[sl] / 2
            out_v[0, sl] = xs - hw0
            out_v[1, sl] = ys - hw1
            out_v[2, sl] = xs + hw0
            out_v[3, sl] = ys + hw1
        pltpu.sync_copy(out_v.at[0], x1o.at[b])
        pltpu.sync_copy(out_v.at[1], y1o.at[b])
        pltpu.sync_copy(out_v.at[2], x2o.at[b])
        pltpu.sync_copy(out_v.at[3], y2o.at[b])
        pltpu.sync_copy(vals_v, sco.at[b])
        for q in range(KPAD // 16):
            sl = pl.ds(q * 16, 16)
            lidx_v[sl] = lidx_v[sl] // HW
        pltpu.sync_copy(lidx_v, clo.at[b])


def _sc_call(scores, rmaxf, cmaxf, regf, whf):
    mesh = plsc.VectorSubcoreMesh(core_axis_name="c", subcore_axis_name="s")
    fn = functools.partial(
        pl.kernel, mesh=mesh,
        out_type=[jax.ShapeDtypeStruct((B, KPAD), jnp.float32)] * 5
        + [jax.ShapeDtypeStruct((B, KPAD), jnp.int32)],
        scratch_types=[
            pltpu.VMEM((NR,), jnp.float32),
            pltpu.VMEM((C,), jnp.float32),
            pltpu.VMEM((NR + 16,), jnp.int32),
            pltpu.VMEM((2 * C + TOPK, W), jnp.float32),
            pltpu.VMEM((KPAD,), jnp.float32),
            pltpu.VMEM((KPAD,), jnp.int32),
            pltpu.VMEM((KPAD,), jnp.int32),
            pltpu.VMEM((KPAD,), jnp.float32),
            pltpu.VMEM((KPAD,), jnp.float32),
            pltpu.VMEM((KPAD,), jnp.float32),
            pltpu.VMEM((KPAD,), jnp.float32),
            pltpu.VMEM((4, KPAD), jnp.float32),
            pltpu.SemaphoreType.DMA,
        ],
        compiler_params=pltpu.CompilerParams(needs_layout_passes=False),
    )(_sc_extract)
    return fn(scores, rmaxf, cmaxf, regf, whf)


@jax.jit
def kernel(cls, reg, wh):
    scores, rmax, cmax = _dense_call(cls)
    rmaxf = rmax.reshape(B, NR)
    cmaxf = cmax.reshape(B, C)
    regf = reg.reshape(-1)
    whf = wh.reshape(-1)
    x1, y1, x2, y2, sc, cl = _sc_call(scores, rmaxf, cmaxf, regf, whf)
    bboxes = jnp.stack(
        [x1[:, :TOPK], y1[:, :TOPK], x2[:, :TOPK], y2[:, :TOPK]], axis=-1)
    scores_out = sc[:, :TOPK].reshape(-1)
    clses = cl[:, :TOPK].reshape(-1)
    return bboxes, scores_out, clses
